# baseline (device time: 35872 ns/iter reference)
import jax
import jax.numpy as jnp
from jax import lax
from jax.experimental import pallas as pl
from jax.experimental.pallas import tpu as pltpu

N_DEV = 4


def kernel(A, B):
    m_per, _k = A.shape
    _, n = B.shape

    def body(a_ref, b_ref, out_ref, send_sems, recv_sems):
        my = lax.axis_index("i")

        barrier_sem = pltpu.get_barrier_semaphore()
        for d in range(1, N_DEV):
            peer = (my + d) % N_DEV
            pl.semaphore_signal(
                barrier_sem,
                inc=1,
                device_id=(peer,),
                device_id_type=pl.DeviceIdType.MESH,
            )
        pl.semaphore_wait(barrier_sem, N_DEV - 1)

        c = jnp.dot(
            a_ref[...].astype(jnp.bfloat16),
            b_ref[...].astype(jnp.bfloat16),
            preferred_element_type=jnp.float32,
        ).astype(jnp.bfloat16)
        out_ref[pl.ds(my * m_per, m_per), :] = c

        sends = []
        for d in range(1, N_DEV):
            peer = (my + d) % N_DEV
            rdma = pltpu.make_async_remote_copy(
                src_ref=out_ref.at[pl.ds(my * m_per, m_per)],
                dst_ref=out_ref.at[pl.ds(my * m_per, m_per)],
                send_sem=send_sems.at[d - 1],
                recv_sem=recv_sems.at[my],
                device_id=(peer,),
                device_id_type=pl.DeviceIdType.MESH,
            )
            rdma.start()
            sends.append(rdma)

        for d in range(1, N_DEV):
            origin = (my + d) % N_DEV
            recv = pltpu.make_async_remote_copy(
                src_ref=out_ref.at[pl.ds(origin * m_per, m_per)],
                dst_ref=out_ref.at[pl.ds(origin * m_per, m_per)],
                send_sem=send_sems.at[d - 1],
                recv_sem=recv_sems.at[origin],
                device_id=(my,),
                device_id_type=pl.DeviceIdType.MESH,
            )
            recv.wait_recv()

        for rdma in sends:
            rdma.wait_send()

    out_shape = jax.ShapeDtypeStruct((N_DEV * m_per, n), jnp.bfloat16)
    return pl.pallas_call(
        body,
        out_shape=out_shape,
        in_specs=[
            pl.BlockSpec(memory_space=pltpu.VMEM),
            pl.BlockSpec(memory_space=pltpu.VMEM),
        ],
        out_specs=pl.BlockSpec(memory_space=pltpu.VMEM),
        scratch_shapes=[
            pltpu.SemaphoreType.DMA((N_DEV - 1,)),
            pltpu.SemaphoreType.DMA((N_DEV,)),
        ],
        compiler_params=pltpu.CompilerParams(collective_id=0),
    )(A, B)


# device time: 25652 ns/iter; 1.3984x vs baseline; 1.3984x over previous
import jax
import jax.numpy as jnp
from jax import lax
from jax.experimental import pallas as pl
from jax.experimental.pallas import tpu as pltpu

N_DEV = 4


def kernel(A, B):
    m_per, k = A.shape
    _, n = B.shape

    def body(a_ref, b_ref, out_ref, comm_ref, send_sems, recv_sems):
        my = lax.axis_index("i")

        barrier_sem = pltpu.get_barrier_semaphore()
        for d in range(1, N_DEV):
            peer = (my + d) % N_DEV
            pl.semaphore_signal(
                barrier_sem,
                inc=1,
                device_id=(peer,),
                device_id_type=pl.DeviceIdType.MESH,
            )
        pl.semaphore_wait(barrier_sem, N_DEV - 1)

        a16 = a_ref[...].astype(jnp.bfloat16)
        comm_ref[my] = a16

        sends = []
        for d in range(1, N_DEV):
            peer = (my + d) % N_DEV
            rdma = pltpu.make_async_remote_copy(
                src_ref=comm_ref.at[my],
                dst_ref=comm_ref.at[my],
                send_sem=send_sems.at[d - 1],
                recv_sem=recv_sems.at[my],
                device_id=(peer,),
                device_id_type=pl.DeviceIdType.MESH,
            )
            rdma.start()
            sends.append(rdma)

        b16 = b_ref[...].astype(jnp.bfloat16)
        out_ref[pl.ds(my * m_per, m_per), :] = jnp.dot(
            a16, b16, preferred_element_type=jnp.float32
        ).astype(jnp.bfloat16)

        for d in (1, 3, 2):
            origin = (my + d) % N_DEV
            recv = pltpu.make_async_remote_copy(
                src_ref=comm_ref.at[origin],
                dst_ref=comm_ref.at[origin],
                send_sem=send_sems.at[d - 1],
                recv_sem=recv_sems.at[origin],
                device_id=(my,),
                device_id_type=pl.DeviceIdType.MESH,
            )
            recv.wait_recv()
            out_ref[pl.ds(origin * m_per, m_per), :] = jnp.dot(
                comm_ref[origin], b16, preferred_element_type=jnp.float32
            ).astype(jnp.bfloat16)

        for rdma in sends:
            rdma.wait_send()

    out_shape = jax.ShapeDtypeStruct((N_DEV * m_per, n), jnp.bfloat16)
    return pl.pallas_call(
        body,
        out_shape=out_shape,
        in_specs=[
            pl.BlockSpec(memory_space=pltpu.VMEM),
            pl.BlockSpec(memory_space=pltpu.VMEM),
        ],
        out_specs=pl.BlockSpec(memory_space=pltpu.VMEM),
        scratch_shapes=[
            pltpu.VMEM((N_DEV, m_per, k), jnp.bfloat16),
            pltpu.SemaphoreType.DMA((N_DEV - 1,)),
            pltpu.SemaphoreType.DMA((N_DEV,)),
        ],
        compiler_params=pltpu.CompilerParams(collective_id=0),
    )(A, B)
